# baseline (speedup 1.0000x reference)
import jax
import jax.numpy as jnp
from jax.experimental import pallas as pl

_B, _G, _H = 8, 64, 256
_P = _B * _G


def _dot(a, b, precision=jax.lax.Precision.DEFAULT):
    return jax.lax.dot_general(
        a, b, (((1,), (0,)), ((), ())),
        precision=precision,
        preferred_element_type=jnp.float32)


def _tdot(a, b, precision=jax.lax.Precision.DEFAULT):
    return jax.lax.dot_general(
        a, b, (((0,), (0,)), ((), ())),
        precision=precision,
        preferred_element_type=jnp.float32)


def _relu(x):
    return jnp.maximum(x, 0.0)


def _fused_kernel(pts_ref, ftsT_ref, grid_ref,
                  pe_W1, pe_b1, pe_W2, pe_b2, pe_W3, pe_b3,
                  fe_W1, fe_b1, fe_W2, fe_b2,
                  rn_W1, rn_b1, rn_W2, rn_b2,
                  cl_W1, cl_b1, cl_W2, cl_b2,
                  po_W1, po_b1, po_W2, po_b2,
                  sz_W1, sz_b1, sz_W2, sz_b2,
                  probs_ref, pose_ref, size_ref, proc_ref, gf_ref):
    pts = pts_ref[...]
    ptsT = pts.T
    pf = _relu(_tdot(pe_W1[...], ptsT) + pe_b1[...])
    pf = _relu(_tdot(pe_W2[...], pf) + pe_b2[...])
    pf = _tdot(pe_W3[...], pf) + pe_b3[...]
    fe = _relu(_tdot(fe_W1[...], ftsT_ref[...]) + fe_b1[...])
    fe = _tdot(fe_W2[...], fe) + fe_b2[...]
    combined = jnp.concatenate([pf, fe], axis=0)

    gridT = grid_ref[...].T
    dx = pts[:, 0:1] - gridT[0:1, :]
    dy = pts[:, 1:2] - gridT[1:2, :]
    dz = pts[:, 2:3] - gridT[2:3, :]
    d = jnp.sqrt(dx * dx + dy * dy + dz * dz)
    dmin = jnp.min(d, axis=1, keepdims=True)
    j_iota = jax.lax.broadcasted_iota(jnp.int32, (_P, _G), 1)
    idx = jnp.min(jnp.where(d == dmin, j_iota, _G), axis=1, keepdims=True)

    p2 = jax.lax.broadcasted_iota(jnp.int32, (_P, _P), 0)
    q2 = jax.lax.broadcasted_iota(jnp.int32, (_P, _P), 1)
    cond = ((p2 >> 6) == (q2 >> 6)) & (idx == (q2 & (_G - 1)))
    win = jnp.max(jnp.where(cond, p2, -1), axis=0, keepdims=True)
    onehot = (p2 == win).astype(jnp.float32)
    gff = _dot(combined, onehot, jax.lax.Precision.HIGHEST)

    h = _relu(_tdot(rn_W1[...], gff) + rn_b1[...])
    procf = _tdot(rn_W2[...], h) + rn_b2[...]

    for b in range(_B):
        cols = slice(b * _G, (b + 1) * _G)
        gf_ref[b] = gff[:, cols]
        proc_ref[b] = procf[:, cols]

    bp = jax.lax.broadcasted_iota(jnp.int32, (_P, _B), 0) >> 6
    bq = jax.lax.broadcasted_iota(jnp.int32, (_P, _B), 1)
    avgT = jnp.where(bp == bq, 1.0 / _G, 0.0).astype(jnp.float32)
    agg = _dot(procf, avgT, jax.lax.Precision.HIGHEST).T

    c1 = _relu(_dot(agg, cl_W1[...]) + cl_b1[...])
    logits = _dot(c1, cl_W2[...]) + cl_b2[...]
    m = jnp.max(logits, axis=1, keepdims=True)
    e = jnp.exp(logits - m)
    probs_ref[...] = e / jnp.sum(e, axis=1, keepdims=True)
    p1 = _relu(_dot(agg, po_W1[...]) + po_b1[...])
    pose_ref[...] = _dot(p1, po_W2[...]) + po_b2[...]
    s1 = _relu(_dot(agg, sz_W1[...]) + sz_b1[...])
    size_ref[...] = jax.nn.sigmoid(_dot(s1, sz_W2[...]) + sz_b2[...])


def kernel(point_cloud, features, grid_points,
           pe_W1, pe_b1, pe_W2, pe_b2, pe_W3, pe_b3,
           fe_W1, fe_b1, fe_W2, fe_b2,
           rn_W1, rn_b1, rn_W2, rn_b2,
           cl_W1, cl_b1, cl_W2, cl_b2,
           po_W1, po_b1, po_W2, po_b2,
           sz_W1, sz_b1, sz_W2, sz_b2):
    C = cl_W2.shape[1]
    pts = point_cloud[:, :_G, :].reshape(_P, 3)
    ftsT = features[:, :_G, :].reshape(_P, 64).T
    pe_b1c, pe_b2c, pe_b3c, fe_b1c, fe_b2c, rn_b1c, rn_b2c = [
        b.reshape(-1, 1) for b in (pe_b1, pe_b2, pe_b3, fe_b1, fe_b2,
                                   rn_b1, rn_b2)]
    cl_b1r, cl_b2r, po_b1r, po_b2r, sz_b1r, sz_b2r = [
        b.reshape(1, -1) for b in (cl_b1, cl_b2, po_b1, po_b2,
                                   sz_b1, sz_b2)]

    out_shape = (
        jax.ShapeDtypeStruct((_B, C), jnp.float32),
        jax.ShapeDtypeStruct((_B, 7), jnp.float32),
        jax.ShapeDtypeStruct((_B, 3), jnp.float32),
        jax.ShapeDtypeStruct((_B, _H, _G), jnp.float32),
        jax.ShapeDtypeStruct((_B, 2 * _H, _G), jnp.float32),
    )
    return pl.pallas_call(
        _fused_kernel, out_shape=out_shape)(
            pts, ftsT, grid_points,
            pe_W1, pe_b1c, pe_W2, pe_b2c, pe_W3, pe_b3c,
            fe_W1, fe_b1c, fe_W2, fe_b2c,
            rn_W1, rn_b1c, rn_W2, rn_b2c,
            cl_W1, cl_b1r, cl_W2, cl_b2r,
            po_W1, po_b1r, po_W2, po_b2r,
            sz_W1, sz_b1r, sz_W2, sz_b2r)


# speedup vs baseline: 1.9781x; 1.9781x over previous
import jax
import jax.numpy as jnp
from jax.experimental import pallas as pl

_B, _G, _H = 8, 64, 256
_P = _B * _G


def _dot(a, b, precision=jax.lax.Precision.DEFAULT):
    return jax.lax.dot_general(
        a, b, (((1,), (0,)), ((), ())),
        precision=precision,
        preferred_element_type=jnp.float32)


def _relu(x):
    return jnp.maximum(x, 0.0)


def _fused_kernel(pts_ref, ptsT_ref, fts_ref, grid_ref,
                  pe_W1, pe_b1, pe_W2, pe_b2, pe_W3, pe_b3,
                  fe_W1, fe_b1, fe_W2, fe_b2,
                  rn_W1, rn_b1, rn_W2, rn_b2,
                  cl_W1, cl_b1, cl_W2, cl_b2,
                  po_W1, po_b1, po_W2, po_b2,
                  sz_W1, sz_b1, sz_W2, sz_b2,
                  probs_ref, pose_ref, size_ref, proc_ref, gf_ref):
    pts = pts_ref[...]
    pf = _relu(_dot(pts, pe_W1[...]) + pe_b1[...])
    pf = _relu(_dot(pf, pe_W2[...]) + pe_b2[...])
    pf = _dot(pf, pe_W3[...]) + pe_b3[...]
    fe = _relu(_dot(fts_ref[...], fe_W1[...]) + fe_b1[...])
    fe = _dot(fe, fe_W2[...]) + fe_b2[...]
    combined = jnp.concatenate([pf, fe], axis=1)

    gx = grid_ref[:, 0:1]
    gy = grid_ref[:, 1:2]
    gz = grid_ref[:, 2:3]
    dx = gx - ptsT_ref[0:1, :]
    dy = gy - ptsT_ref[1:2, :]
    dz = gz - ptsT_ref[2:3, :]
    d = jnp.sqrt(dx * dx + dy * dy + dz * dz)
    dmin = jnp.min(d, axis=0, keepdims=True)
    j_iota = jax.lax.broadcasted_iota(jnp.int32, (_G, _P), 0)
    idx = jnp.min(jnp.where(d == dmin, j_iota, _G), axis=0, keepdims=True)

    q2 = jax.lax.broadcasted_iota(jnp.int32, (_P, _P), 0)
    p2 = jax.lax.broadcasted_iota(jnp.int32, (_P, _P), 1)
    cond = ((q2 >> 6) == (p2 >> 6)) & (idx == (q2 & (_G - 1)))
    win = jnp.max(jnp.where(cond, p2, -1), axis=1, keepdims=True)
    onehot = (p2 == win).astype(jnp.float32)
    gff = _dot(onehot, combined, jax.lax.Precision.HIGHEST)
    gf_ref[...] = gff

    h = _relu(_dot(gff, rn_W1[...]) + rn_b1[...])
    procf = _dot(h, rn_W2[...]) + rn_b2[...]
    proc_ref[...] = procf

    bq = jax.lax.broadcasted_iota(jnp.int32, (_B, _P), 0)
    bp = jax.lax.broadcasted_iota(jnp.int32, (_B, _P), 1) >> 6
    avg = jnp.where(bq == bp, 1.0 / _G, 0.0).astype(jnp.float32)
    agg = _dot(avg, procf, jax.lax.Precision.HIGHEST)

    c1 = _relu(_dot(agg, cl_W1[...]) + cl_b1[...])
    logits = _dot(c1, cl_W2[...]) + cl_b2[...]
    m = jnp.max(logits, axis=1, keepdims=True)
    e = jnp.exp(logits - m)
    probs_ref[...] = e / jnp.sum(e, axis=1, keepdims=True)
    p1 = _relu(_dot(agg, po_W1[...]) + po_b1[...])
    pose_ref[...] = _dot(p1, po_W2[...]) + po_b2[...]
    s1 = _relu(_dot(agg, sz_W1[...]) + sz_b1[...])
    size_ref[...] = jax.nn.sigmoid(_dot(s1, sz_W2[...]) + sz_b2[...])


def kernel(point_cloud, features, grid_points,
           pe_W1, pe_b1, pe_W2, pe_b2, pe_W3, pe_b3,
           fe_W1, fe_b1, fe_W2, fe_b2,
           rn_W1, rn_b1, rn_W2, rn_b2,
           cl_W1, cl_b1, cl_W2, cl_b2,
           po_W1, po_b1, po_W2, po_b2,
           sz_W1, sz_b1, sz_W2, sz_b2):
    C = cl_W2.shape[1]
    pts = point_cloud[:, :_G, :].reshape(_P, 3)
    fts = features[:, :_G, :].reshape(_P, 64)
    ptsT = pts.T

    biases = [pe_b1, pe_b2, pe_b3, fe_b1, fe_b2, rn_b1, rn_b2,
              cl_b1, cl_b2, po_b1, po_b2, sz_b1, sz_b2]
    (pe_b1, pe_b2, pe_b3, fe_b1, fe_b2, rn_b1, rn_b2,
     cl_b1, cl_b2, po_b1, po_b2, sz_b1, sz_b2) = [
        b.reshape(1, -1) for b in biases]

    out_shape = (
        jax.ShapeDtypeStruct((_B, C), jnp.float32),
        jax.ShapeDtypeStruct((_B, 7), jnp.float32),
        jax.ShapeDtypeStruct((_B, 3), jnp.float32),
        jax.ShapeDtypeStruct((_P, _H), jnp.float32),
        jax.ShapeDtypeStruct((_P, 2 * _H), jnp.float32),
    )
    probs, pose, size, procf, gff = pl.pallas_call(
        _fused_kernel, out_shape=out_shape)(
            pts, ptsT, fts, grid_points,
            pe_W1, pe_b1, pe_W2, pe_b2, pe_W3, pe_b3,
            fe_W1, fe_b1, fe_W2, fe_b2,
            rn_W1, rn_b1, rn_W2, rn_b2,
            cl_W1, cl_b1, cl_W2, cl_b2,
            po_W1, po_b1, po_W2, po_b2,
            sz_W1, sz_b1, sz_W2, sz_b2)

    proc = procf.reshape(_B, _G, _H).transpose(0, 2, 1)
    gf = gff.reshape(_B, _G, 2 * _H).transpose(0, 2, 1)
    return (probs, pose, size, proc, gf)
